# baseline (speedup 1.0000x reference)
import functools

import jax
import jax.numpy as jnp
from jax import lax
from jax.experimental import pallas as pl
from jax.experimental.pallas import tpu as pltpu
from jax.experimental.pallas import tpu_sc as plsc

B = 9216
D = 256
NKEYS = 1024
BLK = 512
NBLK = B // BLK

NC, NS = 2, 16
NW = NC * NS
B_PER_W = B // NW
CHUNK = 96
NCHUNK = B_PER_W // CHUNK


def _argmin_body(f_ref, k_ref, out_ref, knorm_ref):
    @pl.when(pl.program_id(0) == 0)
    def _():
        kw0 = k_ref[...]
        knorm_ref[...] = jnp.sum(kw0 * kw0, axis=1)[None, :]

    f = f_ref[...]
    kw = k_ref[...]
    mm = lax.dot_general(f, kw, (((1,), (1,)), ((), ())),
                         preferred_element_type=jnp.float32)
    fnorm = jnp.sum(f * f, axis=1, keepdims=True)
    d = (fnorm + knorm_ref[...]) - 2.0 * mm
    dmin = jnp.min(d, axis=1, keepdims=True)
    ii = lax.broadcasted_iota(jnp.int32, d.shape, 1)
    idx = jnp.min(jnp.where(d == dmin, ii, NKEYS), axis=1)
    out_ref[0, 0, :] = idx


def _argmin_tc(flat, key_weights):
    out = pl.pallas_call(
        _argmin_body,
        grid=(NBLK,),
        in_specs=[
            pl.BlockSpec((BLK, D), lambda i: (i, 0)),
            pl.BlockSpec((NKEYS, D), lambda i: (0, 0)),
        ],
        out_specs=pl.BlockSpec((1, 1, BLK), lambda i: (i, 0, 0)),
        out_shape=jax.ShapeDtypeStruct((NBLK, 1, BLK), jnp.int32),
        scratch_shapes=[pltpu.VMEM((1, NKEYS), jnp.float32)],
    )(flat, key_weights)
    return out.reshape(B)


@functools.cache
def _make_gather_add_sc():
    @functools.partial(
        pl.kernel,
        mesh=plsc.VectorSubcoreMesh(core_axis_name="c", subcore_axis_name="s"),
        out_type=jax.ShapeDtypeStruct((B, D), jnp.float32),
        scratch_types=[
            pltpu.VMEM((CHUNK,), jnp.int32),
            pltpu.VMEM((CHUNK, D), jnp.float32),
            pltpu.VMEM((CHUNK, D), jnp.float32),
            pltpu.SemaphoreType.DMA,
        ],
    )
    def _gather_add_sc(flat_hbm, idx_hbm, val_hbm, out_hbm, idx_v, rows_v,
                       flat_v, sem):
        wid = lax.axis_index("s") * NC + lax.axis_index("c")
        for c in range(NCHUNK):
            base = wid * B_PER_W + c * CHUNK
            pltpu.sync_copy(idx_hbm.at[pl.ds(base, CHUNK)], idx_v)
            gather = pltpu.async_copy(val_hbm.at[idx_v], rows_v, sem)
            pltpu.sync_copy(flat_hbm.at[pl.ds(base, CHUNK)], flat_v)
            gather.wait()

            def add_row(r, _):
                for j in range(D // 16):
                    sl = pl.ds(j * 16, 16)
                    rows_v[r, sl] = rows_v[r, sl] + flat_v[r, sl]
                return ()

            lax.fori_loop(0, CHUNK, add_row, ())
            pltpu.sync_copy(rows_v, out_hbm.at[pl.ds(base, CHUNK)])

    return _gather_add_sc


def kernel(inputs, key_weights, value_weights):
    size = inputs.shape
    flat = inputs.reshape(-1, D)
    idx = _argmin_tc(flat, key_weights)
    out = _make_gather_add_sc()(flat, idx, value_weights)
    return out.reshape(size)


# speedup vs baseline: 1.0213x; 1.0213x over previous
import functools

import jax
import jax.numpy as jnp
from jax import lax
from jax.experimental import pallas as pl
from jax.experimental.pallas import tpu as pltpu
from jax.experimental.pallas import tpu_sc as plsc

B = 9216
D = 256
NKEYS = 1024
BLK = 512
NBLK = B // BLK

NC, NS = 2, 16
NW = NC * NS
B_PER_W = B // NW
CHUNK = 96
NCHUNK = B_PER_W // CHUNK


def _argmin_body(f_ref, k_ref, out_ref, knorm_ref):
    @pl.when(pl.program_id(0) == 0)
    def _():
        kw0 = k_ref[...]
        knorm_ref[...] = jnp.sum(kw0 * kw0, axis=1)[None, :]

    f = f_ref[...]
    kw = k_ref[...]
    mm = lax.dot_general(f, kw, (((1,), (1,)), ((), ())),
                         preferred_element_type=jnp.float32)
    fnorm = jnp.sum(f * f, axis=1, keepdims=True)
    d = (fnorm + knorm_ref[...]) - 2.0 * mm
    dmin = jnp.min(d, axis=1, keepdims=True)
    ii = lax.broadcasted_iota(jnp.int32, d.shape, 1)
    idx = jnp.min(jnp.where(d == dmin, ii, NKEYS), axis=1)
    out_ref[0, 0, :] = idx


def _argmin_tc(flat, key_weights):
    out = pl.pallas_call(
        _argmin_body,
        grid=(NBLK,),
        in_specs=[
            pl.BlockSpec((BLK, D), lambda i: (i, 0)),
            pl.BlockSpec((NKEYS, D), lambda i: (0, 0)),
        ],
        out_specs=pl.BlockSpec((1, 1, BLK), lambda i: (i, 0, 0)),
        out_shape=jax.ShapeDtypeStruct((NBLK, 1, BLK), jnp.int32),
        scratch_shapes=[pltpu.VMEM((1, NKEYS), jnp.float32)],
    )(flat, key_weights)
    return out.reshape(B)


@functools.cache
def _make_gather_add_sc():
    @functools.partial(
        pl.kernel,
        mesh=plsc.VectorSubcoreMesh(core_axis_name="c", subcore_axis_name="s"),
        out_type=jax.ShapeDtypeStruct((B, D), jnp.float32),
        scratch_types=[
            pltpu.VMEM((NCHUNK, CHUNK), jnp.int32),
            pltpu.VMEM((B_PER_W, D), jnp.float32),
            pltpu.VMEM((2, CHUNK, D), jnp.float32),
            pltpu.SemaphoreType.DMA,
            pltpu.SemaphoreType.DMA,
            pltpu.SemaphoreType.DMA,
            pltpu.SemaphoreType.DMA,
        ],
    )
    def _gather_add_sc(flat_hbm, idx_hbm, val_hbm, out_hbm, idx_v, rows_v,
                       flat_v, gsem, fsem0, fsem1, osem):
        wid = lax.axis_index("s") * NC + lax.axis_index("c")
        base_w = wid * B_PER_W
        fsems = (fsem0, fsem1)

        icopies = [
            pltpu.async_copy(idx_hbm.at[pl.ds(base_w + c * CHUNK, CHUNK)],
                             idx_v.at[c], osem)
            for c in range(NCHUNK)
        ]
        for ic in icopies:
            ic.wait()

        gathers = [
            pltpu.async_copy(val_hbm.at[idx_v.at[c]],
                             rows_v.at[pl.ds(c * CHUNK, CHUNK)], gsem)
            for c in range(NCHUNK)
        ]
        flats = [None] * NCHUNK
        for c in range(2):
            flats[c] = pltpu.async_copy(
                flat_hbm.at[pl.ds(base_w + c * CHUNK, CHUNK)],
                flat_v.at[c], fsems[c])
        for g in gathers:
            g.wait()

        owrites = []
        for c in range(NCHUNK):
            b = c % 2
            flats[c].wait()

            @plsc.parallel_loop(0, CHUNK, 1, unroll=4)
            def _add(r):
                rr = c * CHUNK + r
                for j in range(D // 16):
                    sl = pl.ds(j * 16, 16)
                    rows_v[rr, sl] = rows_v[rr, sl] + flat_v[b, r, sl]

            if c + 2 < NCHUNK:
                flats[c + 2] = pltpu.async_copy(
                    flat_hbm.at[pl.ds(base_w + (c + 2) * CHUNK, CHUNK)],
                    flat_v.at[b], fsems[b])
            owrites.append(pltpu.async_copy(
                rows_v.at[pl.ds(c * CHUNK, CHUNK)],
                out_hbm.at[pl.ds(base_w + c * CHUNK, CHUNK)], osem))
        for ow in owrites:
            ow.wait()

    return _gather_add_sc


def kernel(inputs, key_weights, value_weights):
    size = inputs.shape
    flat = inputs.reshape(-1, D)
    idx = _argmin_tc(flat, key_weights)
    out = _make_gather_add_sc()(flat, idx, value_weights)
    return out.reshape(size)
